# baseline (speedup 1.0000x reference)
import functools

import jax
import jax.numpy as jnp
from jax import lax
from jax.experimental import pallas as pl
from jax.experimental.pallas import tpu as pltpu
from jax.experimental.pallas import tpu_sc as plsc

N_NODES = 100000
NUM_GROUPS = 4
BP_ITERS = 5
DW = 8

NC = 2
NS = 16
NW = NC * NS

NPAD = 102400
CHUNK_NODES = 400
EC = 128
ROWS_PER_IT = 8
ROWS_PER_W = 392
DUMMY = N_NODES


def _sc_pass_body(first_pass, half, *refs):
    (mp0, mp1, sinit, betav, e3, zeros,
     parts,
     m0b, m1b, sib, sb, bbuf, idx_v, rows_v, stab_sh, acc_sh,
     gsem, ssem) = refs

    sid = lax.axis_index("s")
    wid = half * NS + sid

    nz = NPAD // NS
    pltpu.sync_copy(zeros.at[pl.ds(sid * nz, nz)], acc_sh.at[pl.ds(sid * nz, nz)])
    pltpu.sync_copy(zeros.at[pl.ds(0, CHUNK_NODES)], sb)

    pltpu.sync_copy(betav, bbuf)
    bvec = bbuf[...]
    iot = lax.iota(jnp.int32, 16)
    n_chunks = (NPAD // NS) // CHUNK_NODES

    def chunk_body(ci, carry):
        n0 = sid * (NPAD // NS) + ci * CHUNK_NODES
        if first_pass:
            pltpu.sync_copy(mp0.at[pl.ds(n0, CHUNK_NODES)], m0b)
        else:
            pltpu.sync_copy(mp0.at[pl.ds(n0, CHUNK_NODES)], m0b)
            pltpu.sync_copy(mp1.at[pl.ds(n0, CHUNK_NODES)], m1b)
        pltpu.sync_copy(sinit.at[pl.ds(n0, CHUNK_NODES)], sib)
        for b in range(CHUNK_NODES // 16):
            idxn = b * 16 + iot
            zs = []
            for g in range(NUM_GROUPS):
                idxg = jnp.full((16,), g, jnp.int32)
                si = plsc.load_gather(sib, [idxn, idxg])
                if first_pass:
                    zs.append(si)
                else:
                    idxg8 = jnp.full((16,), NUM_GROUPS + g, jnp.int32)
                    m = (plsc.load_gather(m0b, [idxn, idxg8]) +
                         plsc.load_gather(m1b, [idxn, idxg8]))
                    zs.append(bvec * m + si)
            mx = jnp.maximum(jnp.maximum(zs[0], zs[1]),
                             jnp.maximum(zs[2], zs[3]))
            es = [jnp.exp(z - mx) for z in zs]
            tot = (es[0] + es[1]) + (es[2] + es[3])
            for g in range(NUM_GROUPS):
                plsc.store_scatter(sb, [idxn, jnp.full((16,), NUM_GROUPS + g,
                                                       jnp.int32)],
                                   es[g] / tot)
            if first_pass:
                for c in range(NUM_GROUPS):
                    idxc = jnp.full((16,), c, jnp.int32)
                    xv = plsc.load_gather(m0b, [idxn, idxc])
                    plsc.store_scatter(sb, [idxn, idxc], xv)
        pltpu.sync_copy(sb, stab_sh.at[pl.ds(n0, CHUNK_NODES)])
        return carry

    lax.fori_loop(0, n_chunks, chunk_body, 0)
    tbl = stab_sh

    plsc.subcore_barrier()

    NB = ROWS_PER_W // ROWS_PER_IT
    base = wid * ROWS_PER_W

    def edge_body(i, carry):
        r0 = base + i * ROWS_PER_IT
        pltpu.sync_copy(e3.at[pl.ds(r0, ROWS_PER_IT)], idx_v)
        gds = [pltpu.async_copy(tbl.at[idx_v.at[j, 0]], rows_v.at[j], gsem)
               for j in range(ROWS_PER_IT)]
        sds = []
        for j in range(ROWS_PER_IT):
            gds[j].wait()
            sds.append(pltpu.async_copy(rows_v.at[j],
                                        acc_sh.at[idx_v.at[j, 1]], ssem,
                                        add=True))
        for d in sds:
            d.wait()
        return carry

    lax.fori_loop(0, NB, edge_body, 0)

    plsc.subcore_barrier()

    pltpu.sync_copy(acc_sh.at[pl.ds(sid * nz, nz)],
                    parts.at[pl.ds(sid * nz, nz)])


def _make_sc_pass(first_pass, half):
    mesh = plsc.VectorSubcoreMesh(core_axis_name="c", subcore_axis_name="s",
                                  num_cores=1, num_subcores=NS)
    out_type = jax.ShapeDtypeStruct((NPAD, DW), jnp.float32)
    scratch = [
        pltpu.VMEM((CHUNK_NODES, DW), jnp.float32),
        pltpu.VMEM((CHUNK_NODES, DW), jnp.float32),
        pltpu.VMEM((CHUNK_NODES, NUM_GROUPS), jnp.float32),
        pltpu.VMEM((CHUNK_NODES, DW), jnp.float32),
        pltpu.VMEM((16,), jnp.float32),
        pltpu.VMEM((ROWS_PER_IT, 2, EC), jnp.int32),
        pltpu.VMEM((ROWS_PER_IT, EC, DW), jnp.float32),
        pltpu.VMEM_SHARED((NPAD, DW), jnp.float32),
        pltpu.VMEM_SHARED((NPAD, DW), jnp.float32),
        pltpu.SemaphoreType.DMA,
        pltpu.SemaphoreType.DMA,
    ]
    return pl.kernel(functools.partial(_sc_pass_body, first_pass, half),
                     out_type=out_type, mesh=mesh, scratch_types=scratch,
                     compiler_params=pltpu.CompilerParams(
                         needs_layout_passes=False,
                         use_tc_tiling_on_sc=False))


def _tc_tail_body(nblocks, sga, sgb, m5a, m5b, sinit, x, beta,
                  wl, bl, wr, fc1w, fc1b, fc2w, fc2b,
                  out_ref, ent_ref, pooled_acc, ent_acc):
    i = pl.program_id(0)

    m5 = m5a[...][:, NUM_GROUPS:] + m5b[...][:, NUM_GROUPS:]
    z = beta[0, 0] * m5 + sinit[...]
    z = z - jnp.max(z, axis=1, keepdims=True)
    e = jnp.exp(z)
    s5 = e / jnp.sum(e, axis=1, keepdims=True)
    entb = jnp.sum(s5 * jnp.log(s5 + 1e-12))

    agg = sga[...] + sgb[...]
    deg = jnp.maximum(agg[:, 3:4], 1.0)
    mean_nbr = agg[:, 0:3] / deg
    h = mean_nbr @ wl[...] + bl[...] + x[...] @ wr[...]
    pooled = lax.dot_general(s5, h, (((0,), (0,)), ((), ())))

    @pl.when(i == 0)
    def _():
        pooled_acc[...] = jnp.zeros_like(pooled_acc)
        ent_acc[0] = 0.0

    pooled_acc[0:4, :] += pooled
    ent_acc[0] += entb

    @pl.when(i == nblocks - 1)
    def _():
        p = pooled_acc[0:4, :]
        v = jnp.concatenate([p[0:1], p[1:2], p[2:3], p[3:4]], axis=1)
        v = jnp.maximum(v @ fc1w[...] + fc1b[...], 0.0)
        out_ref[...] = v @ fc2w[...] + fc2b[...]
        ent_ref[...] = jnp.full((1, 1), -(ent_acc[0] / N_NODES), jnp.float32)


def kernel(x, edge_index, W_l, b_l, W_r, beta, s_init, fc1_W, fc1_b, fc2_W,
           fc2_b):
    n, e = x.shape[0], edge_index.shape[1]
    epad = NW * ROWS_PER_W * EC
    assert e <= epad
    src2 = jnp.full((epad,), DUMMY, jnp.int32).at[:e].set(edge_index[0]
                                                          ).reshape(-1, EC)
    dst2 = jnp.full((epad,), DUMMY, jnp.int32).at[:e].set(edge_index[1]
                                                          ).reshape(-1, EC)
    e3 = jnp.stack([src2, dst2], axis=1)
    xpad = jnp.zeros((NPAD, DW), jnp.float32)
    xpad = xpad.at[:n, 0:3].set(x).at[:n, 3].set(1.0)
    sinit_pad = jnp.zeros((NPAD, NUM_GROUPS), jnp.float32).at[:n].set(s_init)
    zeros_tab = jnp.zeros((NPAD, DW), jnp.float32)
    betav = jnp.full((16,), beta, jnp.float32)

    first_a = _make_sc_pass(True, 0)
    first_b = _make_sc_pass(True, 1)
    bp_a = _make_sc_pass(False, 0)
    bp_b = _make_sc_pass(False, 1)

    sg0 = first_a(xpad, zeros_tab, sinit_pad, betav, e3, zeros_tab)
    sg1 = first_b(xpad, zeros_tab, sinit_pad, betav, e3, zeros_tab)
    p0, p1 = sg0, sg1

    for _ in range(BP_ITERS - 1):
        p0, p1 = (bp_a(p0, p1, sinit_pad, betav, e3, zeros_tab),
                  bp_b(p0, p1, sinit_pad, betav, e3, zeros_tab))

    B = 2000
    nblocks = n // B
    beta11 = jnp.full((1, 1), beta, jnp.float32)
    in_specs = [
        pl.BlockSpec((B, DW), lambda i: (i, 0)),
        pl.BlockSpec((B, DW), lambda i: (i, 0)),
        pl.BlockSpec((B, DW), lambda i: (i, 0)),
        pl.BlockSpec((B, DW), lambda i: (i, 0)),
        pl.BlockSpec((B, NUM_GROUPS), lambda i: (i, 0)),
        pl.BlockSpec((B, 3), lambda i: (i, 0)),
        pl.BlockSpec((1, 1), lambda i: (0, 0)),
        pl.BlockSpec((3, 8), lambda i: (0, 0)),
        pl.BlockSpec((1, 8), lambda i: (0, 0)),
        pl.BlockSpec((3, 8), lambda i: (0, 0)),
        pl.BlockSpec((32, 8), lambda i: (0, 0)),
        pl.BlockSpec((1, 8), lambda i: (0, 0)),
        pl.BlockSpec((8, 6), lambda i: (0, 0)),
        pl.BlockSpec((1, 6), lambda i: (0, 0)),
    ]
    out_specs = [
        pl.BlockSpec((1, 6), lambda i: (0, 0)),
        pl.BlockSpec((1, 1), lambda i: (0, 0)),
    ]
    out2, ent2 = pl.pallas_call(
        functools.partial(_tc_tail_body, nblocks),
        grid=(nblocks,),
        in_specs=in_specs,
        out_specs=out_specs,
        out_shape=[jax.ShapeDtypeStruct((1, 6), jnp.float32),
                   jax.ShapeDtypeStruct((1, 1), jnp.float32)],
        scratch_shapes=[pltpu.VMEM((8, 8), jnp.float32),
                        pltpu.SMEM((1,), jnp.float32)],
    )(sg0[:n], sg1[:n], p0[:n], p1[:n],
      sinit_pad[:n], x, beta11,
      W_l, b_l.reshape(1, 8), W_r, fc1_W, fc1_b.reshape(1, 8),
      fc2_W, fc2_b.reshape(1, 6))
    return out2.reshape(6), ent2[0, 0]


# speedup vs baseline: 1.4635x; 1.4635x over previous
import functools

import jax
import jax.numpy as jnp
from jax import lax
from jax.experimental import pallas as pl
from jax.experimental.pallas import tpu as pltpu
from jax.experimental.pallas import tpu_sc as plsc

N_NODES = 100000
NUM_GROUPS = 4
BP_ITERS = 5
DW = 8

NC = 2
NS = 16
NW = NC * NS

NPAD = 102400
CHUNK_NODES = 400
EC = 128
ROWS_PER_IT = 8
ROWS_PER_W = 392
DUMMY = N_NODES


def _sc_pass_body(first_pass, *refs):
    (mparts, sinit, betav, e3, zeros,
     parts,
     m0b, m1b, sib, sb, bbuf, idx_v, rows_v, stab_sh, acc_sh,
     gsem, ssem) = refs

    cid = lax.axis_index("c")
    sid = lax.axis_index("s")
    wid = sid * NC + cid

    nz = NPAD // NS
    pltpu.sync_copy(zeros.at[pl.ds(sid * nz, nz)], acc_sh.at[pl.ds(sid * nz, nz)])
    pltpu.sync_copy(zeros.at[pl.ds(0, CHUNK_NODES)], sb)

    pltpu.sync_copy(betav, bbuf)
    bvec = bbuf[...]
    iot = lax.iota(jnp.int32, 16)
    n_chunks = (NPAD // NS) // CHUNK_NODES

    def chunk_body(ci, carry):
        n0 = sid * (NPAD // NS) + ci * CHUNK_NODES
        ds = [pltpu.async_copy(mparts.at[0, pl.ds(n0, CHUNK_NODES)], m0b,
                               gsem),
              pltpu.async_copy(sinit.at[pl.ds(n0, CHUNK_NODES)], sib, gsem)]
        if not first_pass:
            ds.append(pltpu.async_copy(mparts.at[1, pl.ds(n0, CHUNK_NODES)],
                                       m1b, gsem))
        for d in ds:
            d.wait()
        for b in range(CHUNK_NODES // 16):
            idxn = b * 16 + iot
            zs = []
            for g in range(NUM_GROUPS):
                idxg = jnp.full((16,), g, jnp.int32)
                si = plsc.load_gather(sib, [idxn, idxg])
                if first_pass:
                    zs.append(si)
                else:
                    idxg8 = jnp.full((16,), NUM_GROUPS + g, jnp.int32)
                    m = (plsc.load_gather(m0b, [idxn, idxg8]) +
                         plsc.load_gather(m1b, [idxn, idxg8]))
                    zs.append(bvec * m + si)
            mx = jnp.maximum(jnp.maximum(zs[0], zs[1]),
                             jnp.maximum(zs[2], zs[3]))
            es = [jnp.exp(z - mx) for z in zs]
            tot = (es[0] + es[1]) + (es[2] + es[3])
            for g in range(NUM_GROUPS):
                plsc.store_scatter(sb, [idxn, jnp.full((16,), NUM_GROUPS + g,
                                                       jnp.int32)],
                                   es[g] / tot)
            if first_pass:
                for c in range(NUM_GROUPS):
                    idxc = jnp.full((16,), c, jnp.int32)
                    xv = plsc.load_gather(m0b, [idxn, idxc])
                    plsc.store_scatter(sb, [idxn, idxc], xv)
        pltpu.sync_copy(sb, stab_sh.at[pl.ds(n0, CHUNK_NODES)])
        return carry

    lax.fori_loop(0, n_chunks, chunk_body, 0)
    tbl = stab_sh

    plsc.subcore_barrier()

    NB = ROWS_PER_W // ROWS_PER_IT
    base = wid * ROWS_PER_W

    def edge_body(i, carry):
        r0 = base + i * ROWS_PER_IT
        pltpu.sync_copy(e3.at[pl.ds(r0, ROWS_PER_IT)], idx_v)
        gds = [pltpu.async_copy(tbl.at[idx_v.at[j, 0]], rows_v.at[j], gsem)
               for j in range(ROWS_PER_IT)]
        sds = []
        for j in range(ROWS_PER_IT):
            gds[j].wait()
            sds.append(pltpu.async_copy(rows_v.at[j],
                                        acc_sh.at[idx_v.at[j, 1]], ssem,
                                        add=True))
        for d in sds:
            d.wait()
        return carry

    lax.fori_loop(0, NB, edge_body, 0)

    plsc.subcore_barrier()

    pltpu.sync_copy(acc_sh.at[pl.ds(sid * nz, nz)],
                    parts.at[cid, pl.ds(sid * nz, nz)])


def _make_sc_pass(first_pass):
    mesh = plsc.VectorSubcoreMesh(core_axis_name="c", subcore_axis_name="s",
                                  num_cores=NC, num_subcores=NS)
    out_type = jax.ShapeDtypeStruct((NC, NPAD, DW), jnp.float32)
    scratch = [
        pltpu.VMEM((CHUNK_NODES, DW), jnp.float32),
        pltpu.VMEM((CHUNK_NODES, DW), jnp.float32),
        pltpu.VMEM((CHUNK_NODES, NUM_GROUPS), jnp.float32),
        pltpu.VMEM((CHUNK_NODES, DW), jnp.float32),
        pltpu.VMEM((16,), jnp.float32),
        pltpu.VMEM((ROWS_PER_IT, 2, EC), jnp.int32),
        pltpu.VMEM((ROWS_PER_IT, EC, DW), jnp.float32),
        pltpu.VMEM_SHARED((NPAD, DW), jnp.float32),
        pltpu.VMEM_SHARED((NPAD, DW), jnp.float32),
        pltpu.SemaphoreType.DMA,
        pltpu.SemaphoreType.DMA,
    ]
    return pl.kernel(functools.partial(_sc_pass_body, first_pass),
                     out_type=out_type, mesh=mesh, scratch_types=scratch,
                     compiler_params=pltpu.CompilerParams(
                         needs_layout_passes=False,
                         use_tc_tiling_on_sc=False))


def _tc_tail_body(nblocks, sga, sgb, m5a, m5b, sinit, x, beta,
                  wl, bl, wr, fc1w, fc1b, fc2w, fc2b,
                  out_ref, ent_ref, pooled_acc, ent_acc):
    i = pl.program_id(0)

    m5 = m5a[...][:, NUM_GROUPS:] + m5b[...][:, NUM_GROUPS:]
    z = beta[0, 0] * m5 + sinit[...]
    z = z - jnp.max(z, axis=1, keepdims=True)
    e = jnp.exp(z)
    s5 = e / jnp.sum(e, axis=1, keepdims=True)
    entb = jnp.sum(s5 * jnp.log(s5 + 1e-12))

    agg = sga[...] + sgb[...]
    deg = jnp.maximum(agg[:, 3:4], 1.0)
    mean_nbr = agg[:, 0:3] / deg
    h = mean_nbr @ wl[...] + bl[...] + x[...] @ wr[...]
    pooled = lax.dot_general(s5, h, (((0,), (0,)), ((), ())))

    @pl.when(i == 0)
    def _():
        pooled_acc[...] = jnp.zeros_like(pooled_acc)
        ent_acc[0] = 0.0

    pooled_acc[0:4, :] += pooled
    ent_acc[0] += entb

    @pl.when(i == nblocks - 1)
    def _():
        p = pooled_acc[0:4, :]
        v = jnp.concatenate([p[0:1], p[1:2], p[2:3], p[3:4]], axis=1)
        v = jnp.maximum(v @ fc1w[...] + fc1b[...], 0.0)
        out_ref[...] = v @ fc2w[...] + fc2b[...]
        ent_ref[...] = jnp.full((1, 1), -(ent_acc[0] / N_NODES), jnp.float32)


def kernel(x, edge_index, W_l, b_l, W_r, beta, s_init, fc1_W, fc1_b, fc2_W,
           fc2_b):
    n, e = x.shape[0], edge_index.shape[1]
    epad = NW * ROWS_PER_W * EC
    assert e <= epad
    src2 = jnp.full((epad,), DUMMY, jnp.int32).at[:e].set(edge_index[0]
                                                          ).reshape(-1, EC)
    dst2 = jnp.full((epad,), DUMMY, jnp.int32).at[:e].set(edge_index[1]
                                                          ).reshape(-1, EC)
    e3 = jnp.stack([src2, dst2], axis=1)
    xpad = jnp.zeros((NC, NPAD, DW), jnp.float32)
    xpad = xpad.at[0, :n, 0:3].set(x).at[0, :n, 3].set(1.0)
    sinit_pad = jnp.zeros((NPAD, NUM_GROUPS), jnp.float32).at[:n].set(s_init)
    zeros_tab = jnp.zeros((NPAD, DW), jnp.float32)
    betav = jnp.full((16,), beta, jnp.float32)

    first_k = _make_sc_pass(True)
    bp_k = _make_sc_pass(False)

    sage_parts = first_k(xpad, sinit_pad, betav, e3, zeros_tab)
    parts = sage_parts

    for _ in range(BP_ITERS - 1):
        parts = bp_k(parts, sinit_pad, betav, e3, zeros_tab)

    B = 2000
    nblocks = n // B
    beta11 = jnp.full((1, 1), beta, jnp.float32)
    in_specs = [
        pl.BlockSpec((B, DW), lambda i: (i, 0)),
        pl.BlockSpec((B, DW), lambda i: (i, 0)),
        pl.BlockSpec((B, DW), lambda i: (i, 0)),
        pl.BlockSpec((B, DW), lambda i: (i, 0)),
        pl.BlockSpec((B, NUM_GROUPS), lambda i: (i, 0)),
        pl.BlockSpec((B, 3), lambda i: (i, 0)),
        pl.BlockSpec((1, 1), lambda i: (0, 0)),
        pl.BlockSpec((3, 8), lambda i: (0, 0)),
        pl.BlockSpec((1, 8), lambda i: (0, 0)),
        pl.BlockSpec((3, 8), lambda i: (0, 0)),
        pl.BlockSpec((32, 8), lambda i: (0, 0)),
        pl.BlockSpec((1, 8), lambda i: (0, 0)),
        pl.BlockSpec((8, 6), lambda i: (0, 0)),
        pl.BlockSpec((1, 6), lambda i: (0, 0)),
    ]
    out_specs = [
        pl.BlockSpec((1, 6), lambda i: (0, 0)),
        pl.BlockSpec((1, 1), lambda i: (0, 0)),
    ]
    out2, ent2 = pl.pallas_call(
        functools.partial(_tc_tail_body, nblocks),
        grid=(nblocks,),
        in_specs=in_specs,
        out_specs=out_specs,
        out_shape=[jax.ShapeDtypeStruct((1, 6), jnp.float32),
                   jax.ShapeDtypeStruct((1, 1), jnp.float32)],
        scratch_shapes=[pltpu.VMEM((8, 8), jnp.float32),
                        pltpu.SMEM((1,), jnp.float32)],
    )(sage_parts[0, :n], sage_parts[1, :n], parts[0, :n], parts[1, :n],
      sinit_pad[:n], x, beta11,
      W_l, b_l.reshape(1, 8), W_r, fc1_W, fc1_b.reshape(1, 8),
      fc2_W, fc2_b.reshape(1, 6))
    return out2.reshape(6), ent2[0, 0]
